# baseline (speedup 1.0000x reference)
import functools

import jax
import jax.numpy as jnp
from jax import lax
from jax.experimental import pallas as pl
from jax.experimental.pallas import tpu as pltpu
from jax.experimental.pallas import tpu_sc as plsc

_ALPHA = 0.5
_NEG_LAMBDA = 1.0

_BSC = 1024
_NW = 32
_W = _BSC // _NW


def _tc_main_body(x_ref, y_ref, m_ref, out_ref, *, Bb, K, L, CH):
    x = x_ref[...]
    yb = y_ref[0]

    e16 = jnp.exp(-x).astype(jnp.bfloat16)
    e2 = e16.reshape(Bb * K, L)
    m = m_ref[...]
    s = jnp.concatenate(
        [jax.lax.dot_general(m, e2[c * CH * K:(c + 1) * CH * K],
                             (((1,), (0,)), ((), ())),
                             preferred_element_type=jnp.float32)
         for c in range(Bb // CH)], axis=0)

    col3 = jax.lax.broadcasted_iota(jnp.int32, (Bb, K, L), 2)
    xsel = jnp.where(col3 == yb.reshape(Bb, 1, 1), x, jnp.inf)
    posk = jnp.min(xsel, axis=2)
    pos = jnp.min(posk, axis=1, keepdims=True)

    col = jax.lax.broadcasted_iota(jnp.int32, (Bb, L), 1)
    tmask = col == yb

    t = jnp.maximum(_ALPHA + pos + _NEG_LAMBDA * jnp.log(s), 0.0)
    partial = jnp.sum(jnp.where(tmask, 0.0, t))

    @pl.when(pl.program_id(0) == 0)
    def _init():
        out_ref[0, 0] = 0.0

    out_ref[0, 0] += partial


def _tc_tail_body(s_ref, xmin_ref, y_ref, out_ref, *, Bsc, L):
    s = s_ref[...]
    xmin = xmin_ref[...]
    yb = y_ref[0]
    col = jax.lax.broadcasted_iota(jnp.int32, (Bsc, L), 1)
    tmask = col == yb
    pos = jnp.min(jnp.where(tmask, xmin, jnp.inf), axis=1, keepdims=True)
    t = jnp.maximum(_ALPHA + pos + _NEG_LAMBDA * jnp.log(s), 0.0)
    out_ref[0, 0] = jnp.sum(jnp.where(tmask, 0.0, t))


def _sc_body(x_hbm, s_hbm, xmin_hbm, xbuf, sbuf, mbuf, *, BT, K, L):
    wid = lax.axis_index("s") * 2 + lax.axis_index("c")
    base = BT + wid * _W
    ob = wid * _W
    pltpu.sync_copy(x_hbm.at[pl.ds(base, _W)], xbuf)

    def row_fn(r, carry):
        for j in range(L // 16):
            acc = jnp.zeros((16,), jnp.float32)
            mn = jnp.full((16,), jnp.inf, jnp.float32)
            for k in range(K):
                chunk = xbuf[r, k, pl.ds(j * 16, 16)]
                acc = acc + jnp.exp(-chunk)
                mn = jnp.minimum(mn, chunk)
            sbuf[r, pl.ds(j * 16, 16)] = acc
            mbuf[r, pl.ds(j * 16, 16)] = mn
        return carry

    lax.fori_loop(0, _W, row_fn, 0)

    pltpu.sync_copy(sbuf, s_hbm.at[pl.ds(ob, _W)])
    pltpu.sync_copy(mbuf, xmin_hbm.at[pl.ds(ob, _W)])


def kernel(y_hat, y):
    B, L, K = y_hat.shape
    BT = B - _BSC
    Bb = 1024
    CH = 64
    G = BT // Bb
    x_t = jnp.transpose(y_hat, (0, 2, 1))

    mesh = plsc.VectorSubcoreMesh(core_axis_name="c", subcore_axis_name="s")
    sc_call = pl.kernel(
        functools.partial(_sc_body, BT=BT, K=K, L=L),
        mesh=mesh,
        out_type=[
            jax.ShapeDtypeStruct((_BSC, L), jnp.float32),
            jax.ShapeDtypeStruct((_BSC, L), jnp.float32),
        ],
        scratch_types=[
            pltpu.VMEM((_W, K, L), jnp.float32),
            pltpu.VMEM((_W, L), jnp.float32),
            pltpu.VMEM((_W, L), jnp.float32),
        ],
    )
    s_sc, xmin_sc = sc_call(x_t)

    y3 = y[:BT].reshape(G, Bb, 1)
    m = (jnp.arange(CH * K, dtype=jnp.int32)[None, :] // K
         == jnp.arange(CH, dtype=jnp.int32)[:, None]).astype(jnp.bfloat16)
    p1 = pl.pallas_call(
        functools.partial(_tc_main_body, Bb=Bb, K=K, L=L, CH=CH),
        grid=(G,),
        in_specs=[
            pl.BlockSpec((Bb, K, L), lambda i: (i, 0, 0)),
            pl.BlockSpec((1, Bb, 1), lambda i: (i, 0, 0)),
            pl.BlockSpec((CH, CH * K), lambda i: (0, 0)),
        ],
        out_specs=pl.BlockSpec(memory_space=pltpu.SMEM),
        out_shape=jax.ShapeDtypeStruct((1, 1), jnp.float32),
    )(x_t, y3, m)

    p2 = pl.pallas_call(
        functools.partial(_tc_tail_body, Bsc=_BSC, L=L),
        in_specs=[
            pl.BlockSpec((_BSC, L), lambda: (0, 0)),
            pl.BlockSpec((_BSC, L), lambda: (0, 0)),
            pl.BlockSpec((1, _BSC, 1), lambda: (0, 0, 0)),
        ],
        out_specs=pl.BlockSpec(memory_space=pltpu.SMEM),
        out_shape=jax.ShapeDtypeStruct((1, 1), jnp.float32),
    )(s_sc, xmin_sc, y[BT:].reshape(1, _BSC, 1))

    return (p1[0, 0] + p2[0, 0]) * (1.0 / (B * (L - 1)))


# speedup vs baseline: 1.0390x; 1.0390x over previous
import functools

import jax
import jax.numpy as jnp
from jax import lax
from jax.experimental import pallas as pl
from jax.experimental.pallas import tpu as pltpu
from jax.experimental.pallas import tpu_sc as plsc

_ALPHA = 0.5
_NEG_LAMBDA = 1.0

_BSC = 1024
_NW = 32
_W = _BSC // _NW


def _tc_main_body(x_ref, y_ref, m_ref, out_ref, *, Bb, K, L, CH):
    x = x_ref[...]
    yb = y_ref[0]

    e16 = jnp.exp(-x).astype(jnp.bfloat16)
    e2 = e16.reshape(Bb * K, L)
    m = m_ref[...]
    s = jnp.concatenate(
        [jax.lax.dot_general(m, e2[c * CH * K:(c + 1) * CH * K],
                             (((1,), (0,)), ((), ())),
                             preferred_element_type=jnp.float32)
         for c in range(Bb // CH)], axis=0)

    col3 = jax.lax.broadcasted_iota(jnp.int32, (Bb, K, L), 2)
    xsel = jnp.where(col3 == yb.reshape(Bb, 1, 1), x, jnp.inf)
    posk = jnp.min(xsel, axis=2)
    pos = jnp.min(posk, axis=1, keepdims=True)

    col = jax.lax.broadcasted_iota(jnp.int32, (Bb, L), 1)
    tmask = col == yb

    t = jnp.maximum(_ALPHA + pos + _NEG_LAMBDA * jnp.log(s), 0.0)
    partial = jnp.sum(jnp.where(tmask, 0.0, t))

    @pl.when(pl.program_id(0) == 0)
    def _init():
        out_ref[0, 0] = 0.0

    out_ref[0, 0] += partial


def _tc_tail_body(s_ref, xmin_ref, y_ref, p1_ref, out_ref, *, Bsc, L,
                  inv_count):
    s = s_ref[...]
    xmin = xmin_ref[...]
    yb = y_ref[0]
    col = jax.lax.broadcasted_iota(jnp.int32, (Bsc, L), 1)
    tmask = col == yb
    pos = jnp.min(jnp.where(tmask, xmin, jnp.inf), axis=1, keepdims=True)
    t = jnp.maximum(_ALPHA + pos + _NEG_LAMBDA * jnp.log(s), 0.0)
    p2 = jnp.sum(jnp.where(tmask, 0.0, t))
    out_ref[0, 0] = (p1_ref[0, 0] + p2) * inv_count


def _sc_body(x_hbm, s_hbm, xmin_hbm, xbuf, sbuf, mbuf, *, BT, K, L):
    wid = lax.axis_index("s") * 2 + lax.axis_index("c")
    base = BT + wid * _W
    ob = wid * _W
    pltpu.sync_copy(x_hbm.at[pl.ds(base, _W)], xbuf)

    def row_fn(r, carry):
        for j in range(L // 16):
            acc = jnp.zeros((16,), jnp.float32)
            mn = jnp.full((16,), jnp.inf, jnp.float32)
            for k in range(K):
                chunk = xbuf[r, k, pl.ds(j * 16, 16)]
                acc = acc + jnp.exp(-chunk)
                mn = jnp.minimum(mn, chunk)
            sbuf[r, pl.ds(j * 16, 16)] = acc
            mbuf[r, pl.ds(j * 16, 16)] = mn
        return carry

    lax.fori_loop(0, _W, row_fn, 0)

    pltpu.sync_copy(sbuf, s_hbm.at[pl.ds(ob, _W)])
    pltpu.sync_copy(mbuf, xmin_hbm.at[pl.ds(ob, _W)])


def kernel(y_hat, y):
    B, L, K = y_hat.shape
    BT = B - _BSC
    Bb = 1024
    CH = 64
    G = BT // Bb
    x_t = jnp.transpose(y_hat, (0, 2, 1))

    mesh = plsc.VectorSubcoreMesh(core_axis_name="c", subcore_axis_name="s")
    sc_call = pl.kernel(
        functools.partial(_sc_body, BT=BT, K=K, L=L),
        mesh=mesh,
        out_type=[
            jax.ShapeDtypeStruct((_BSC, L), jnp.float32),
            jax.ShapeDtypeStruct((_BSC, L), jnp.float32),
        ],
        scratch_types=[
            pltpu.VMEM((_W, K, L), jnp.float32),
            pltpu.VMEM((_W, L), jnp.float32),
            pltpu.VMEM((_W, L), jnp.float32),
        ],
    )
    s_sc, xmin_sc = sc_call(x_t)

    y3 = y[:BT].reshape(G, Bb, 1)
    m = (jnp.arange(CH * K, dtype=jnp.int32)[None, :] // K
         == jnp.arange(CH, dtype=jnp.int32)[:, None]).astype(jnp.bfloat16)
    p1 = pl.pallas_call(
        functools.partial(_tc_main_body, Bb=Bb, K=K, L=L, CH=CH),
        grid=(G,),
        in_specs=[
            pl.BlockSpec((Bb, K, L), lambda i: (i, 0, 0)),
            pl.BlockSpec((1, Bb, 1), lambda i: (i, 0, 0)),
            pl.BlockSpec((CH, CH * K), lambda i: (0, 0)),
        ],
        out_specs=pl.BlockSpec(memory_space=pltpu.SMEM),
        out_shape=jax.ShapeDtypeStruct((1, 1), jnp.float32),
    )(x_t, y3, m)

    out = pl.pallas_call(
        functools.partial(_tc_tail_body, Bsc=_BSC, L=L,
                          inv_count=1.0 / (B * (L - 1))),
        in_specs=[
            pl.BlockSpec((_BSC, L), lambda: (0, 0)),
            pl.BlockSpec((_BSC, L), lambda: (0, 0)),
            pl.BlockSpec((1, _BSC, 1), lambda: (0, 0, 0)),
            pl.BlockSpec(memory_space=pltpu.SMEM),
        ],
        out_specs=pl.BlockSpec(memory_space=pltpu.SMEM),
        out_shape=jax.ShapeDtypeStruct((1, 1), jnp.float32),
    )(s_sc, xmin_sc, y[BT:].reshape(1, _BSC, 1), p1)

    return out[0, 0]


# speedup vs baseline: 1.8813x; 1.8107x over previous
import functools

import jax
import jax.numpy as jnp
from jax.experimental import pallas as pl
from jax.experimental.pallas import tpu as pltpu

_ALPHA = 0.5
_NEG_LAMBDA = 1.0


def _loss_body(x_ref, y_ref, m_ref, out_ref, *, Bb, K, L, CH, inv_count,
               num_blocks):
    x = x_ref[...]
    yb = y_ref[0]

    e16 = jnp.exp(-x).astype(jnp.bfloat16)
    e2 = e16.reshape(Bb * K, L)
    m = m_ref[...]
    s = jnp.concatenate(
        [jax.lax.dot_general(m, e2[c * CH * K:(c + 1) * CH * K],
                             (((1,), (0,)), ((), ())),
                             preferred_element_type=jnp.float32)
         for c in range(Bb // CH)], axis=0)

    col3 = jax.lax.broadcasted_iota(jnp.int32, (Bb, K, L), 2)
    xsel = jnp.where(col3 == yb.reshape(Bb, 1, 1), x, jnp.inf)
    posk = jnp.min(xsel, axis=2)
    pos = jnp.min(posk, axis=1, keepdims=True)

    col = jax.lax.broadcasted_iota(jnp.int32, (Bb, L), 1)
    tmask = col == yb

    t = jnp.maximum(_ALPHA + pos + _NEG_LAMBDA * jnp.log(s), 0.0)
    partial = jnp.sum(jnp.where(tmask, 0.0, t))

    @pl.when(pl.program_id(0) == 0)
    def _init():
        out_ref[0, 0] = 0.0

    out_ref[0, 0] += partial

    @pl.when(pl.program_id(0) == num_blocks - 1)
    def _finish():
        out_ref[0, 0] = out_ref[0, 0] * inv_count


def kernel(y_hat, y):
    B, L, K = y_hat.shape
    Bb = 1024
    CH = 64
    G = B // Bb
    x_t = jnp.transpose(y_hat, (0, 2, 1))
    y3 = y.reshape(G, Bb, 1)
    m = (jnp.arange(CH * K, dtype=jnp.int32)[None, :] // K
         == jnp.arange(CH, dtype=jnp.int32)[:, None]).astype(jnp.bfloat16)
    total = pl.pallas_call(
        functools.partial(_loss_body, Bb=Bb, K=K, L=L, CH=CH,
                          inv_count=1.0 / (B * (L - 1)), num_blocks=G),
        grid=(G,),
        in_specs=[
            pl.BlockSpec((Bb, K, L), lambda i: (i, 0, 0)),
            pl.BlockSpec((1, Bb, 1), lambda i: (i, 0, 0)),
            pl.BlockSpec((CH, CH * K), lambda i: (0, 0)),
        ],
        out_specs=pl.BlockSpec(memory_space=pltpu.SMEM),
        out_shape=jax.ShapeDtypeStruct((1, 1), jnp.float32),
    )(x_t, y3, m)
    return total[0, 0]
